# baseline (speedup 1.0000x reference)
import functools

import jax
import jax.numpy as jnp
from jax import lax
from jax.experimental import pallas as pl
from jax.experimental.pallas import tpu as pltpu
from jax.experimental.pallas import tpu_sc as plsc

N_FIELDS = 26
VOCAB = 100000
EMB_DIM = 32
BATCH = 16384

ROWS = BATCH * N_FIELDS
NUM_CORES = 2
NUM_SUBCORES = 16
NW = NUM_CORES * NUM_SUBCORES
ROWS_PER_W = ROWS // NW
IDX_W = 128
G = 8
BLOCK = G * IDX_W
NBLK = ROWS_PER_W // BLOCK

VB = 8192
NSQ = VB // 512
NVB = -(-VOCAB // VB)
SQ_F = NVB * NSQ
RPF = SQ_F * 128
SROWS = N_FIELDS * RPF * 4


def _relayout_body(t_ref, o_ref):
    x = t_ref[0]
    for k in range(NSQ):
        xa = x[:, 512 * k:512 * (k + 1)]
        sq = jnp.concatenate(
            [xa[:, 128 * a:128 * (a + 1)] for a in range(4)], axis=0
        )
        o_ref[0, k] = sq.T


_relayout = pl.pallas_call(
    _relayout_body,
    grid=(N_FIELDS, NVB),
    in_specs=[pl.BlockSpec((1, EMB_DIM, VB), lambda f, t: (f, 0, t))],
    out_specs=pl.BlockSpec((1, NSQ, 128, 128), lambda f, t: (f, t, 0, 0)),
    out_shape=jax.ShapeDtypeStruct((N_FIELDS, SQ_F, 128, 128), jnp.float32),
)


def _make_gather():
    mesh = plsc.VectorSubcoreMesh(core_axis_name="c", subcore_axis_name="s")

    @functools.partial(
        pl.kernel,
        mesh=mesh,
        compiler_params=pltpu.CompilerParams(use_tc_tiling_on_sc=False),
        out_type=jax.ShapeDtypeStruct((ROWS, EMB_DIM), jnp.float32),
        scratch_types=[
            pltpu.VMEM((G, IDX_W), jnp.int32),
            pltpu.VMEM((BLOCK, EMB_DIM), jnp.float32),
            pltpu.SemaphoreType.DMA,
        ],
    )
    def gather_kernel(tab_hbm, idx_hbm, out_hbm, idx_v, rows_v, sem):
        wid = lax.axis_index("s") * NUM_CORES + lax.axis_index("c")
        row0 = wid * ROWS_PER_W
        iblk0 = row0 // IDX_W

        def body(blk, carry):
            base = row0 + blk * BLOCK
            irow = pl.multiple_of(iblk0 + blk * G, 8)
            pltpu.sync_copy(idx_hbm.at[pl.ds(irow, G), :], idx_v)
            handles = [
                pltpu.async_copy(
                    tab_hbm.at[idx_v.at[g]],
                    rows_v.at[pl.ds(g * IDX_W, IDX_W)],
                    sem,
                )
                for g in range(G)
            ]
            for h in handles:
                h.wait()
            pltpu.sync_copy(rows_v, out_hbm.at[pl.ds(base, BLOCK)])
            return carry

        lax.fori_loop(0, NBLK, body, 0)

    return gather_kernel


_gather = _make_gather()


@jax.jit
def kernel(x, tables):
    tab_lin = _relayout(tables.transpose(0, 2, 1)).reshape(SROWS, EMB_DIM)
    v = x.astype(jnp.int32)
    f = jnp.arange(N_FIELDS, dtype=jnp.int32)[None, :]
    rows = 4 * (f * RPF + 128 * (v // 512) + (v % 128)) + (v // 128) % 4
    flat_idx = rows.reshape(ROWS // IDX_W, IDX_W)
    out = _gather(tab_lin, flat_idx)
    return out.reshape(BATCH, N_FIELDS * EMB_DIM)


# speedup vs baseline: 1.3109x; 1.3109x over previous
import functools

import jax
import jax.numpy as jnp
from jax import lax
from jax.experimental import pallas as pl
from jax.experimental.pallas import tpu as pltpu
from jax.experimental.pallas import tpu_sc as plsc

N_FIELDS = 26
VOCAB = 100000
EMB_DIM = 32
BATCH = 16384

ROWS = BATCH * N_FIELDS
NUM_CORES = 2
NUM_SUBCORES = 16
NW = NUM_CORES * NUM_SUBCORES
ROWS_PER_W = ROWS // NW
IDX_W = 128
G = 8
BLOCK = G * IDX_W
NBLK = ROWS_PER_W // BLOCK

VB = 20480
NSQ = VB // 512
NVB = -(-VOCAB // VB)
SQ_F = NVB * NSQ
RPF = SQ_F * 128
SROWS = N_FIELDS * RPF * 4


def _relayout_body(t_ref, o_ref):
    x = t_ref[0]
    for k in range(NSQ):
        xa = x[:, 512 * k:512 * (k + 1)]
        sq = jnp.concatenate(
            [xa[:, 128 * a:128 * (a + 1)] for a in range(4)], axis=0
        )
        o_ref[0, k] = sq.T


_relayout = pl.pallas_call(
    _relayout_body,
    grid=(N_FIELDS, NVB),
    in_specs=[pl.BlockSpec((1, EMB_DIM, VB), lambda f, t: (f, 0, t))],
    out_specs=pl.BlockSpec((1, NSQ, 128, 128), lambda f, t: (f, t, 0, 0)),
    out_shape=jax.ShapeDtypeStruct((N_FIELDS, SQ_F, 128, 128), jnp.float32),
)


def _make_gather():
    mesh = plsc.VectorSubcoreMesh(core_axis_name="c", subcore_axis_name="s")

    @functools.partial(
        pl.kernel,
        mesh=mesh,
        compiler_params=pltpu.CompilerParams(use_tc_tiling_on_sc=False),
        out_type=jax.ShapeDtypeStruct((ROWS, EMB_DIM), jnp.float32),
        scratch_types=[
            pltpu.VMEM((G, IDX_W), jnp.int32),
            pltpu.VMEM((BLOCK, EMB_DIM), jnp.float32),
            pltpu.SemaphoreType.DMA,
        ],
    )
    def gather_kernel(tab_hbm, idx_hbm, out_hbm, idx_v, rows_v, sem):
        wid = lax.axis_index("s") * NUM_CORES + lax.axis_index("c")
        row0 = wid * ROWS_PER_W
        iblk0 = row0 // IDX_W

        def body(blk, carry):
            base = row0 + blk * BLOCK
            irow = pl.multiple_of(iblk0 + blk * G, 8)
            pltpu.sync_copy(idx_hbm.at[pl.ds(irow, G), :], idx_v)
            handles = [
                pltpu.async_copy(
                    tab_hbm.at[idx_v.at[g]],
                    rows_v.at[pl.ds(g * IDX_W, IDX_W)],
                    sem,
                )
                for g in range(G)
            ]
            for h in handles:
                h.wait()
            pltpu.sync_copy(rows_v, out_hbm.at[pl.ds(base, BLOCK)])
            return carry

        lax.fori_loop(0, NBLK, body, 0)

    return gather_kernel


_gather = _make_gather()


@jax.jit
def kernel(x, tables):
    tab_lin = _relayout(tables.transpose(0, 2, 1)).reshape(SROWS, EMB_DIM)
    v = x.astype(jnp.int32)
    f = jnp.arange(N_FIELDS, dtype=jnp.int32)[None, :]
    rows = 4 * (f * RPF + 128 * (v // 512) + (v % 128)) + (v // 128) % 4
    flat_idx = rows.reshape(ROWS // IDX_W, IDX_W)
    out = _gather(tab_lin, flat_idx)
    return out.reshape(BATCH, N_FIELDS * EMB_DIM)


# speedup vs baseline: 1.3517x; 1.0311x over previous
import functools

import jax
import jax.numpy as jnp
from jax import lax
from jax.experimental import pallas as pl
from jax.experimental.pallas import tpu as pltpu
from jax.experimental.pallas import tpu_sc as plsc

N_FIELDS = 26
VOCAB = 100000
EMB_DIM = 32
BATCH = 16384

ROWS = BATCH * N_FIELDS
NUM_CORES = 2
NUM_SUBCORES = 16
NW = NUM_CORES * NUM_SUBCORES
ROWS_PER_W = ROWS // NW
IDX_W = 128
G = 8
BLOCK = G * IDX_W
NBLK = ROWS_PER_W // BLOCK

VB = 25600
NSQ = VB // 512
NVB = -(-VOCAB // VB)
SQ_F = NVB * NSQ
RPF = SQ_F * 128
SROWS = N_FIELDS * RPF * 4


def _relayout_body(t_ref, o_ref):
    x = t_ref[0]
    for k in range(NSQ):
        xa = x[:, 512 * k:512 * (k + 1)]
        sq = jnp.concatenate(
            [xa[:, 128 * a:128 * (a + 1)] for a in range(4)], axis=0
        )
        o_ref[0, k] = sq.T


_relayout = pl.pallas_call(
    _relayout_body,
    grid=(N_FIELDS, NVB),
    in_specs=[pl.BlockSpec((1, EMB_DIM, VB), lambda f, t: (f, 0, t))],
    out_specs=pl.BlockSpec((1, NSQ, 128, 128), lambda f, t: (f, t, 0, 0)),
    out_shape=jax.ShapeDtypeStruct((N_FIELDS, SQ_F, 128, 128), jnp.float32),
)


def _make_gather():
    mesh = plsc.VectorSubcoreMesh(core_axis_name="c", subcore_axis_name="s")

    @functools.partial(
        pl.kernel,
        mesh=mesh,
        compiler_params=pltpu.CompilerParams(use_tc_tiling_on_sc=False),
        out_type=jax.ShapeDtypeStruct((ROWS, EMB_DIM), jnp.float32),
        scratch_types=[
            pltpu.VMEM((G, IDX_W), jnp.int32),
            pltpu.VMEM((BLOCK, EMB_DIM), jnp.float32),
            pltpu.SemaphoreType.DMA,
        ],
    )
    def gather_kernel(tab_hbm, idx_hbm, out_hbm, idx_v, rows_v, sem):
        wid = lax.axis_index("s") * NUM_CORES + lax.axis_index("c")
        row0 = wid * ROWS_PER_W
        iblk0 = row0 // IDX_W

        def body(blk, carry):
            base = row0 + blk * BLOCK
            irow = pl.multiple_of(iblk0 + blk * G, 8)
            pltpu.sync_copy(idx_hbm.at[pl.ds(irow, G), :], idx_v)
            handles = [
                pltpu.async_copy(
                    tab_hbm.at[idx_v.at[g]],
                    rows_v.at[pl.ds(g * IDX_W, IDX_W)],
                    sem,
                )
                for g in range(G)
            ]
            for h in handles:
                h.wait()
            pltpu.sync_copy(rows_v, out_hbm.at[pl.ds(base, BLOCK)])
            return carry

        lax.fori_loop(0, NBLK, body, 0)

    return gather_kernel


_gather = _make_gather()


@jax.jit
def kernel(x, tables):
    tab_lin = _relayout(tables.transpose(0, 2, 1)).reshape(SROWS, EMB_DIM)
    v = x.astype(jnp.int32)
    f = jnp.arange(N_FIELDS, dtype=jnp.int32)[None, :]
    rows = 4 * (f * RPF + 128 * (v // 512) + (v % 128)) + (v // 128) % 4
    flat_idx = rows.reshape(ROWS // IDX_W, IDX_W)
    out = _gather(tab_lin, flat_idx)
    return out.reshape(BATCH, N_FIELDS * EMB_DIM)


# speedup vs baseline: 1.4027x; 1.0378x over previous
import functools

import jax
import jax.numpy as jnp
from jax import lax
from jax.experimental import pallas as pl
from jax.experimental.pallas import tpu as pltpu
from jax.experimental.pallas import tpu_sc as plsc

N_FIELDS = 26
VOCAB = 100000
EMB_DIM = 32
BATCH = 16384

ROWS = BATCH * N_FIELDS
NUM_CORES = 2
NUM_SUBCORES = 16
NW = NUM_CORES * NUM_SUBCORES
ROWS_PER_W = ROWS // NW
IDX_W = 128
G = 8
BLOCK = G * IDX_W
NBLK = ROWS_PER_W // BLOCK

VB = 50176
NSQ = VB // 512
NVB = -(-VOCAB // VB)
SQ_F = NVB * NSQ
RPF = SQ_F * 128
SROWS = N_FIELDS * RPF * 4


def _relayout_body(t_ref, o_ref):
    x = t_ref[0]
    for k in range(NSQ):
        xa = x[:, 512 * k:512 * (k + 1)]
        sq = jnp.concatenate(
            [xa[:, 128 * a:128 * (a + 1)] for a in range(4)], axis=0
        )
        o_ref[0, k] = sq.T


_relayout = pl.pallas_call(
    _relayout_body,
    grid=(N_FIELDS, NVB),
    in_specs=[pl.BlockSpec((1, EMB_DIM, VB), lambda f, t: (f, 0, t))],
    out_specs=pl.BlockSpec((1, NSQ, 128, 128), lambda f, t: (f, t, 0, 0)),
    out_shape=jax.ShapeDtypeStruct((N_FIELDS, SQ_F, 128, 128), jnp.float32),
)


def _make_gather():
    mesh = plsc.VectorSubcoreMesh(core_axis_name="c", subcore_axis_name="s")

    @functools.partial(
        pl.kernel,
        mesh=mesh,
        compiler_params=pltpu.CompilerParams(use_tc_tiling_on_sc=False),
        out_type=jax.ShapeDtypeStruct((ROWS, EMB_DIM), jnp.float32),
        scratch_types=[
            pltpu.VMEM((G, IDX_W), jnp.int32),
            pltpu.VMEM((BLOCK, EMB_DIM), jnp.float32),
            pltpu.SemaphoreType.DMA,
        ],
    )
    def gather_kernel(tab_hbm, idx_hbm, out_hbm, idx_v, rows_v, sem):
        wid = lax.axis_index("s") * NUM_CORES + lax.axis_index("c")
        row0 = wid * ROWS_PER_W
        iblk0 = row0 // IDX_W

        def body(blk, carry):
            base = row0 + blk * BLOCK
            irow = pl.multiple_of(iblk0 + blk * G, 8)
            pltpu.sync_copy(idx_hbm.at[pl.ds(irow, G), :], idx_v)
            handles = [
                pltpu.async_copy(
                    tab_hbm.at[idx_v.at[g]],
                    rows_v.at[pl.ds(g * IDX_W, IDX_W)],
                    sem,
                )
                for g in range(G)
            ]
            for h in handles:
                h.wait()
            pltpu.sync_copy(rows_v, out_hbm.at[pl.ds(base, BLOCK)])
            return carry

        lax.fori_loop(0, NBLK, body, 0)

    return gather_kernel


_gather = _make_gather()


@jax.jit
def kernel(x, tables):
    tab_lin = _relayout(tables.transpose(0, 2, 1)).reshape(SROWS, EMB_DIM)
    v = x.astype(jnp.int32)
    f = jnp.arange(N_FIELDS, dtype=jnp.int32)[None, :]
    rows = 4 * (f * RPF + 128 * (v // 512) + (v % 128)) + (v // 128) % 4
    flat_idx = rows.reshape(ROWS // IDX_W, IDX_W)
    out = _gather(tab_lin, flat_idx)
    return out.reshape(BATCH, N_FIELDS * EMB_DIM)
